# baseline (speedup 1.0000x reference)
import functools

import jax
import jax.numpy as jnp
from jax.experimental import pallas as pl
from jax.experimental.pallas import tpu as pltpu

_VMEM_LIMIT = int(0.9 * 64 * 1024 * 1024)
_W0_CHUNKS = 4


def _mlp_kernel(x_ref, b0_ref, b1_ref, w0_hbm, w1_hbm, o_ref,
                w0_v, w1_v, sems, *, tm):
    i = pl.program_id(1)
    d_in = x_ref.shape[1]
    ck = d_in // _W0_CHUNKS

    @pl.when(i == 0)
    def _first_step():
        w1_cp = pltpu.make_async_copy(w1_hbm, w1_v, sems.at[_W0_CHUNKS])
        w1_cp.start()
        cps = []
        for c in range(_W0_CHUNKS):
            cp = pltpu.make_async_copy(
                w0_hbm.at[pl.ds(c * ck, ck), :],
                w0_v.at[pl.ds(c * ck, ck), :],
                sems.at[c])
            cp.start()
            cps.append(cp)
        h = x_ref[...].astype(jnp.bfloat16)
        y = b0_ref[...].astype(jnp.float32)
        for c in range(_W0_CHUNKS):
            cps[c].wait()
            y = y + jnp.dot(h[:, c * ck:(c + 1) * ck],
                            w0_v[pl.ds(c * ck, ck), :],
                            preferred_element_type=jnp.float32)
        y = y * jax.nn.sigmoid(y)
        h2 = y.astype(jnp.bfloat16)
        w1_cp.wait()
        z = jnp.dot(h2, w1_v[...], preferred_element_type=jnp.float32)
        o_ref[...] = z + b1_ref[...]

    @pl.when(i > 0)
    def _steady_step():
        h = x_ref[...].astype(jnp.bfloat16)
        y = jnp.dot(h, w0_v[...], preferred_element_type=jnp.float32)
        y = y + b0_ref[...]
        y = y * jax.nn.sigmoid(y)
        h2 = y.astype(jnp.bfloat16)
        z = jnp.dot(h2, w1_v[...], preferred_element_type=jnp.float32)
        o_ref[...] = z + b1_ref[...]


def kernel(x, w0, b0, w1, b1, *, tm=1024):
    B, d_in = x.shape
    d_in2, d_h = w0.shape
    d_h2, d_out = w1.shape
    assert d_in == d_in2 and d_h == d_h2
    steps = B // tm
    assert steps % 2 == 0

    const = lambda c, i: (0, 0)
    wkw = {"pipeline_mode": pl.Buffered(1)}
    return pl.pallas_call(
        functools.partial(_mlp_kernel, tm=tm),
        out_shape=jax.ShapeDtypeStruct((B, d_out), x.dtype),
        grid=(2, steps // 2),
        in_specs=[
            pl.BlockSpec((tm, d_in), lambda c, i: (c * (B // tm // 2) + i, 0)),
            pl.BlockSpec((1, d_h), const, **wkw),
            pl.BlockSpec((1, d_out), const, **wkw),
            pl.BlockSpec(memory_space=pltpu.MemorySpace.HBM),
            pl.BlockSpec(memory_space=pltpu.MemorySpace.HBM),
        ],
        out_specs=pl.BlockSpec((tm, d_out),
                               lambda c, i: (c * (B // tm // 2) + i, 0)),
        scratch_shapes=[
            pltpu.VMEM((d_in, d_h), jnp.bfloat16),
            pltpu.VMEM((d_h, d_out), jnp.bfloat16),
            pltpu.SemaphoreType.DMA((_W0_CHUNKS + 1,)),
        ],
        compiler_params=pltpu.CompilerParams(
            dimension_semantics=("parallel", "arbitrary"),
            vmem_limit_bytes=_VMEM_LIMIT,
        ),
    )(x, b0, b1, w0, w1)
